# baseline (speedup 1.0000x reference)
import jax
import jax.numpy as jnp
from jax import lax
from jax.experimental import pallas as pl
from jax.experimental.pallas import tpu as pltpu
from jax.experimental.pallas import tpu_sc as plsc

B_, H_, W_, C_ = 4, 384, 384, 96
FLAT = B_ * H_ * W_ * C_
N = B_ * (H_ // 2) * (W_ // 2) * C_
LGB = 20
BSZ = 1 << LGB
NB = FLAT // BSZ
NC, NS = 2, 16
NW = NC * NS
NPW = N // NW
WIN = 8192
NWIN = NPW // WIN
CHUNK = 2048
CHTOT = CHUNK * NS
CAP = N + NB * CHTOT
NBH = NB // NC

_MESH = plsc.VectorSubcoreMesh(
    core_axis_name="c", subcore_axis_name="s", num_cores=NC, num_subcores=NS)
_PARAMS = pltpu.CompilerParams(needs_layout_passes=False)

LANE = lambda: jnp.arange(16, dtype=jnp.int32)


def _hist_kernel(idx1, hist, idxwin, lhist):
    c = lax.axis_index("c")
    s = lax.axis_index("s")
    w = c * NS + s
    lane = LANE()
    zi = jnp.zeros((16,), jnp.int32)
    ones = jnp.ones((16,), jnp.int32)
    for r in range(64):
        lhist[pl.ds(r * 16, 16)] = zi

    def wbody(j, _):
        off = pl.multiple_of(w * NPW + j * WIN, WIN)
        pltpu.sync_copy(idx1.at[pl.ds(off, WIN)], idxwin)

        def rbody(i, _):
            for cc in range(8):
                v = idxwin[pl.ds(i * 128 + cc * 16, 16)]
                a = lax.shift_right_logical(v, LGB) * 16 + lane
                cnt = plsc.load_gather(lhist, [a])
                plsc.store_scatter(lhist, [a], cnt + ones)
            return 0

        lax.fori_loop(0, WIN // 128, rbody, 0)
        return 0

    lax.fori_loop(0, NWIN, wbody, 0)
    pltpu.sync_copy(lhist, hist.at[w])


def _part_kernel(idx1, val1, hist, bidx, bval,
                 idxwin, valwin, destb, lowb, histv, totals, bs, gcur,
                 sem1, sem2):
    c = lax.axis_index("c")
    s = lax.axis_index("s")
    w = c * NS + s
    lane = LANE()
    z16 = jnp.zeros((16,), jnp.int32)
    pltpu.sync_copy(hist, histv)

    def dbody(d, _):
        acc = z16
        accb = z16
        for wp in range(NW):
            row = histv[wp, pl.ds(d * 16, 16)]
            acc = acc + row
            m = jnp.where(wp < w, jnp.int32(1), jnp.int32(0))
            accb = accb + row * m
        own = histv[w, pl.ds(d * 16, 16)]
        exl = plsc.cumsum(own) - own
        totals[d] = jnp.sum(acc)
        gcur[pl.ds(d * 16, 16)] = exl + jnp.sum(accb)
        return 0

    lax.fori_loop(0, NB, dbody, 0)

    def bbody(d, carry):
        bs[d] = carry
        t = totals[d]
        cap = jnp.bitwise_and(t + (CHTOT - 1), jnp.int32(~(CHTOT - 1)))
        return carry + cap

    lax.fori_loop(0, NB, bbody, jnp.int32(0))

    def gbody(d, _):
        gcur[pl.ds(d * 16, 16)] = gcur[pl.ds(d * 16, 16)] + bs[d]
        return 0

    lax.fori_loop(0, NB, gbody, 0)

    def wbody(j, _):
        off = pl.multiple_of(w * NPW + j * WIN, WIN)
        pltpu.sync_copy(idx1.at[pl.ds(off, WIN)], idxwin)
        pltpu.sync_copy(val1.at[pl.ds(off, WIN)], valwin)

        def rbody(i, _):
            for cc in range(8):
                v = idxwin[pl.ds(i * 128 + cc * 16, 16)]
                a = lax.shift_right_logical(v, LGB) * 16 + lane
                p = plsc.load_gather(gcur, [a])
                plsc.store_scatter(gcur, [a], p + 1)
                destb[pl.ds(i * 128 + cc * 16, 16)] = p
                lowb[pl.ds(i * 128 + cc * 16, 16)] = jnp.bitwise_and(
                    v, jnp.int32(BSZ - 1))
            return 0

        lax.fori_loop(0, WIN // 128, rbody, 0)
        cd1 = pltpu.async_copy(lowb, bidx.at[destb], sem1)
        cd2 = pltpu.async_copy(valwin, bval.at[destb], sem2)
        cd1.wait()
        cd2.wait()
        return 0

    lax.fori_loop(0, NWIN, wbody, 0)


def _accum_kernel(bidx1, bval1, hist, out1,
                  histv, totals, bs, zeros, idxch, valch, spmem, semc):
    c = lax.axis_index("c")
    s = lax.axis_index("s")
    lane = LANE()
    z16 = jnp.zeros((16,), jnp.int32)
    zf = jnp.zeros((16,), jnp.float32)
    pltpu.sync_copy(hist, histv)

    def dbody(d, _):
        acc = z16
        for wp in range(NW):
            acc = acc + histv[wp, pl.ds(d * 16, 16)]
        totals[d] = jnp.sum(acc)
        return 0

    lax.fori_loop(0, NB, dbody, 0)

    def bbody(d, carry):
        bs[d] = carry
        t = totals[d]
        cap = jnp.bitwise_and(t + (CHTOT - 1), jnp.int32(~(CHTOT - 1)))
        return carry + cap

    lax.fori_loop(0, NB, bbody, jnp.int32(0))

    for r in range(256):
        zeros[pl.ds(r * 16, 16)] = zf

    def bucket(jb, _):
        b = c * NBH + jb
        cnt = totals[b]
        base = bs[b]
        nch = lax.shift_right_logical(cnt + (CHTOT - 1), 15)
        for t in range(16):
            pltpu.sync_copy(
                zeros,
                spmem.at[pl.ds(pl.multiple_of(s * 65536 + t * 4096, 4096),
                               4096)])
        plsc.subcore_barrier()

        def chunk(t, _):
            loff = (t * NS + s) * CHUNK
            off = pl.multiple_of(base + loff, CHUNK)
            pltpu.sync_copy(bidx1.at[pl.ds(off, CHUNK)], idxch)
            pltpu.sync_copy(bval1.at[pl.ds(off, CHUNK)], valch)

            @pl.when(loff + CHUNK > cnt)
            def _():
                for i in range(16):
                    for cc in range(8):
                        e0 = i * 128 + cc * 16
                        m = (loff + e0 + lane) < cnt
                        iv = idxch[pl.ds(e0, 16)]
                        vv = valch[pl.ds(e0, 16)]
                        idxch[pl.ds(e0, 16)] = jnp.where(m, iv, e0 + lane)
                        valch[pl.ds(e0, 16)] = jnp.where(m, vv, 0.0)

            pltpu.async_copy(valch, spmem.at[idxch], semc, add=True).wait()
            return 0

        lax.fori_loop(0, nch, chunk, 0)
        plsc.subcore_barrier()
        pltpu.sync_copy(
            spmem.at[pl.ds(pl.multiple_of(s * 65536, 65536), 65536)],
            out1.at[pl.ds(pl.multiple_of(b * BSZ + s * 65536, 65536), 65536)])
        plsc.subcore_barrier()
        return 0

    lax.fori_loop(0, NBH, bucket, 0)


def kernel(inputs, output_shape, unpool_mat):
    del output_shape
    idx1 = unpool_mat.reshape(N)
    val1 = inputs.reshape(N)

    hist = pl.kernel(
        _hist_kernel,
        out_type=jax.ShapeDtypeStruct((NW, 1024), jnp.int32),
        mesh=_MESH,
        compiler_params=_PARAMS,
        scratch_types=[
            pltpu.VMEM((WIN,), jnp.int32),
            pltpu.VMEM((1024,), jnp.int32),
        ],
    )(idx1)

    bidx, bval = pl.kernel(
        _part_kernel,
        out_type=(jax.ShapeDtypeStruct((CAP,), jnp.int32),
                  jax.ShapeDtypeStruct((CAP,), jnp.float32)),
        mesh=_MESH,
        compiler_params=_PARAMS,
        scratch_types=[
            pltpu.VMEM((WIN,), jnp.int32),
            pltpu.VMEM((WIN,), jnp.float32),
            pltpu.VMEM((WIN,), jnp.int32),
            pltpu.VMEM((WIN,), jnp.int32),
            pltpu.VMEM((NW, 1024), jnp.int32),
            pltpu.SMEM((64,), jnp.int32),
            pltpu.SMEM((64,), jnp.int32),
            pltpu.VMEM((1024,), jnp.int32),
            pltpu.SemaphoreType.DMA,
            pltpu.SemaphoreType.DMA,
        ],
    )(idx1, val1, hist)

    out1 = pl.kernel(
        _accum_kernel,
        out_type=jax.ShapeDtypeStruct((FLAT,), jnp.float32),
        mesh=_MESH,
        compiler_params=_PARAMS,
        scratch_types=[
            pltpu.VMEM((NW, 1024), jnp.int32),
            pltpu.SMEM((64,), jnp.int32),
            pltpu.SMEM((64,), jnp.int32),
            pltpu.VMEM((4096,), jnp.float32),
            pltpu.VMEM((CHUNK,), jnp.int32),
            pltpu.VMEM((CHUNK,), jnp.float32),
            pltpu.VMEM_SHARED((BSZ,), jnp.float32),
            pltpu.SemaphoreType.DMA,
        ],
    )(bidx, bval, hist)

    return out1.reshape(B_, H_, W_, C_)


# speedup vs baseline: 10.6190x; 10.6190x over previous
import jax
import jax.numpy as jnp
from jax import lax
from jax.experimental import pallas as pl
from jax.experimental.pallas import tpu as pltpu
from jax.experimental.pallas import tpu_sc as plsc

B_, H_, W_, C_ = 4, 384, 384, 96
FLAT = B_ * H_ * W_ * C_
N = B_ * (H_ // 2) * (W_ // 2) * C_
LGB = 20
BSZ = 1 << LGB
NB = FLAT // BSZ
NC, NS = 2, 16
NW = NC * NS
NPW = N // NW
WIN = 8192
NWIN = NPW // WIN
CHUNK = 2048
CHTOT = CHUNK * NS
CAP = N + NW * NWIN * NB * 64 + NB * CHTOT
NBH = NB // NC
SBUF = 12288

_MESH = plsc.VectorSubcoreMesh(
    core_axis_name="c", subcore_axis_name="s", num_cores=NC, num_subcores=NS)
_PARAMS = pltpu.CompilerParams(needs_layout_passes=False)

LANE = lambda: jnp.arange(16, dtype=jnp.int32)


def _hist_kernel(idx1, hist, idxwin, whist, paccv):
    c = lax.axis_index("c")
    s = lax.axis_index("s")
    w = c * NS + s
    lane = LANE()
    zi = jnp.zeros((16,), jnp.int32)
    ones = jnp.ones((16,), jnp.int32)
    m0 = lane < 1
    for r in range(4):
        paccv[pl.ds(r * 16, 16)] = zi

    def wbody(j, _):
        off = pl.multiple_of(w * NPW + j * WIN, WIN)
        pltpu.sync_copy(idx1.at[pl.ds(off, WIN)], idxwin)
        for r in range(NB):
            whist[pl.ds(r * 16, 16)] = zi

        def rbody(i, _):
            for cc in range(8):
                v = idxwin[pl.ds(i * 128 + cc * 16, 16)]
                a = lax.shift_right_logical(v, LGB) * 16 + lane
                cnt = plsc.load_gather(whist, [a])
                plsc.store_scatter(whist, [a], cnt + ones)
            return 0

        lax.fori_loop(0, WIN // 128, rbody, 0)

        def dbody(d, _):
            wcnt = jnp.sum(whist[pl.ds(d * 16, 16)])
            pad = jnp.bitwise_and(wcnt + 63, jnp.int32(~63))
            di = jnp.full((16,), 0, jnp.int32) + d
            cur = plsc.load_gather(paccv, [di])
            plsc.store_scatter(paccv, [di], cur + pad, mask=m0)
            return 0

        lax.fori_loop(0, NB, dbody, 0)
        return 0

    lax.fori_loop(0, NWIN, wbody, 0)
    pltpu.sync_copy(paccv, hist.at[w])


def _part_kernel(idx1, val1, hist, bidx, bval,
                 idxwin, valwin, sidx, sval, whist, lcur, histv,
                 gw, wc, nc_ref, sem1, sem2):
    c = lax.axis_index("c")
    s = lax.axis_index("s")
    w = c * NS + s
    lane = LANE()
    zi = jnp.zeros((16,), jnp.int32)
    ones = jnp.ones((16,), jnp.int32)
    pltpu.sync_copy(hist, histv)

    carry = jnp.int32(0)
    for g in range(4):
        tot = zi
        bef = zi
        for wp in range(NW):
            row = histv[wp, pl.ds(g * 16, 16)]
            tot = tot + row
            bef = bef + row * jnp.where(wp < w, jnp.int32(1), jnp.int32(0))
        caps = jnp.bitwise_and(tot + (CHTOT - 1), jnp.int32(~(CHTOT - 1)))
        gwvec = plsc.cumsum(caps) - caps + carry + bef
        for l in range(16):
            gw[g * 16 + l] = gwvec[l]
        carry = carry + jnp.sum(caps)

    def wbody(j, _):
        off = pl.multiple_of(w * NPW + j * WIN, WIN)
        pltpu.sync_copy(idx1.at[pl.ds(off, WIN)], idxwin)
        pltpu.sync_copy(val1.at[pl.ds(off, WIN)], valwin)
        for r in range(NB):
            whist[pl.ds(r * 16, 16)] = zi

        def rbody(i, _):
            for cc in range(8):
                v = idxwin[pl.ds(i * 128 + cc * 16, 16)]
                a = lax.shift_right_logical(v, LGB) * 16 + lane
                cnt = plsc.load_gather(whist, [a])
                plsc.store_scatter(whist, [a], cnt + ones)
            return 0

        lax.fori_loop(0, WIN // 128, rbody, 0)

        def sbody(d, lstart):
            row = whist[pl.ds(d * 16, 16)]
            wcnt = jnp.sum(row)
            exl = plsc.cumsum(row) - row
            lcur[pl.ds(d * 16, 16)] = exl + lstart
            wc[d] = wcnt
            end = lstart + wcnt
            for k in range(4):
                pos = end + k * 16 + lane
                plsc.store_scatter(sidx, [pos],
                                   jnp.bitwise_and(pos, jnp.int32(1023)))
                plsc.store_scatter(sval, [pos], jnp.zeros((16,), jnp.float32))
            return lstart + jnp.bitwise_and(wcnt + 63, jnp.int32(~63))

        lax.fori_loop(0, NB, sbody, jnp.int32(0))

        def obody(i, _):
            for cc in range(8):
                v = idxwin[pl.ds(i * 128 + cc * 16, 16)]
                x = valwin[pl.ds(i * 128 + cc * 16, 16)]
                a = lax.shift_right_logical(v, LGB) * 16 + lane
                p = plsc.load_gather(lcur, [a])
                plsc.store_scatter(lcur, [a], p + 1)
                plsc.store_scatter(sidx, [p],
                                   jnp.bitwise_and(v, jnp.int32(BSZ - 1)))
                plsc.store_scatter(sval, [p], x)
            return 0

        lax.fori_loop(0, WIN // 128, obody, 0)

        def fbody(d, carry):
            lstart, nc = carry
            n64 = lax.shift_right_logical(wc[d] + 63, 6)
            g0 = gw[d]

            def cbody(k, _):
                so = pl.multiple_of(lstart + k * 64, 64)
                go = pl.multiple_of(g0 + k * 64, 64)
                pltpu.async_copy(sidx.at[pl.ds(so, 64)],
                                 bidx.at[pl.ds(go, 64)], sem1)
                pltpu.async_copy(sval.at[pl.ds(so, 64)],
                                 bval.at[pl.ds(go, 64)], sem2)
                return 0

            lax.fori_loop(0, n64, cbody, 0)
            gw[d] = g0 + n64 * 64
            return (lstart + n64 * 64, nc + n64)

        _, nc = lax.fori_loop(0, NB, fbody, (jnp.int32(0), jnp.int32(0)))

        def drain(k, _):
            pltpu.make_async_copy(
                bidx.at[pl.ds(0, 64)], sidx.at[pl.ds(0, 64)], sem1).wait()
            pltpu.make_async_copy(
                bval.at[pl.ds(0, 64)], sval.at[pl.ds(0, 64)], sem2).wait()
            return 0

        lax.fori_loop(0, nc, drain, 0)
        return 0

    lax.fori_loop(0, NWIN, wbody, 0)


def _accum_kernel(bidx1, bval1, hist, out1,
                  histv, totals, bs, zeros, idxch, valch, spmem, semc):
    c = lax.axis_index("c")
    s = lax.axis_index("s")
    lane = LANE()
    zf = jnp.zeros((16,), jnp.float32)
    pltpu.sync_copy(hist, histv)

    zi = jnp.zeros((16,), jnp.int32)
    carry = jnp.int32(0)
    for g in range(4):
        tot = zi
        for wp in range(NW):
            tot = tot + histv[wp, pl.ds(g * 16, 16)]
        caps = jnp.bitwise_and(tot + (CHTOT - 1), jnp.int32(~(CHTOT - 1)))
        pref = plsc.cumsum(caps) - caps + carry
        for l in range(16):
            totals[g * 16 + l] = tot[l]
            bs[g * 16 + l] = pref[l]
        carry = carry + jnp.sum(caps)

    for r in range(256):
        zeros[pl.ds(r * 16, 16)] = zf

    def bucket(jb, _):
        b = c * NBH + jb
        cnt = totals[b]
        base = bs[b]
        nch = lax.shift_right_logical(cnt + (CHTOT - 1), 15)
        for t in range(16):
            pltpu.sync_copy(
                zeros,
                spmem.at[pl.ds(pl.multiple_of(s * 65536 + t * 4096, 4096),
                               4096)])
        plsc.subcore_barrier()

        def chunk(t, _):
            loff = (t * NS + s) * CHUNK
            off = pl.multiple_of(base + loff, CHUNK)
            pltpu.sync_copy(bidx1.at[pl.ds(off, CHUNK)], idxch)
            pltpu.sync_copy(bval1.at[pl.ds(off, CHUNK)], valch)

            @pl.when(loff + CHUNK > cnt)
            def _():
                for i in range(16):
                    for cc in range(8):
                        e0 = i * 128 + cc * 16
                        m = (loff + e0 + lane) < cnt
                        iv = idxch[pl.ds(e0, 16)]
                        vv = valch[pl.ds(e0, 16)]
                        idxch[pl.ds(e0, 16)] = jnp.where(m, iv, e0 + lane)
                        valch[pl.ds(e0, 16)] = jnp.where(m, vv, 0.0)

            pltpu.async_copy(valch, spmem.at[idxch], semc, add=True).wait()
            return 0

        lax.fori_loop(0, nch, chunk, 0)
        plsc.subcore_barrier()
        pltpu.sync_copy(
            spmem.at[pl.ds(pl.multiple_of(s * 65536, 65536), 65536)],
            out1.at[pl.ds(pl.multiple_of(b * BSZ + s * 65536, 65536), 65536)])
        plsc.subcore_barrier()
        return 0

    lax.fori_loop(0, NBH, bucket, 0)


def kernel(inputs, output_shape, unpool_mat):
    del output_shape
    idx1 = unpool_mat.reshape(N)
    val1 = inputs.reshape(N)

    hist = pl.kernel(
        _hist_kernel,
        out_type=jax.ShapeDtypeStruct((NW, 64), jnp.int32),
        mesh=_MESH,
        compiler_params=_PARAMS,
        scratch_types=[
            pltpu.VMEM((WIN,), jnp.int32),
            pltpu.VMEM((NB * 16,), jnp.int32),
            pltpu.VMEM((64,), jnp.int32),
        ],
    )(idx1)

    bidx, bval = pl.kernel(
        _part_kernel,
        out_type=(jax.ShapeDtypeStruct((CAP,), jnp.int32),
                  jax.ShapeDtypeStruct((CAP,), jnp.float32)),
        mesh=_MESH,
        compiler_params=_PARAMS,
        scratch_types=[
            pltpu.VMEM((WIN,), jnp.int32),
            pltpu.VMEM((WIN,), jnp.float32),
            pltpu.VMEM((SBUF,), jnp.int32),
            pltpu.VMEM((SBUF,), jnp.float32),
            pltpu.VMEM((NB * 16,), jnp.int32),
            pltpu.VMEM((NB * 16,), jnp.int32),
            pltpu.VMEM((NW, 64), jnp.int32),
            pltpu.SMEM((64,), jnp.int32),
            pltpu.SMEM((64,), jnp.int32),
            pltpu.SMEM((4,), jnp.int32),
            pltpu.SemaphoreType.DMA,
            pltpu.SemaphoreType.DMA,
        ],
    )(idx1, val1, hist)

    out1 = pl.kernel(
        _accum_kernel,
        out_type=jax.ShapeDtypeStruct((FLAT,), jnp.float32),
        mesh=_MESH,
        compiler_params=_PARAMS,
        scratch_types=[
            pltpu.VMEM((NW, 64), jnp.int32),
            pltpu.SMEM((64,), jnp.int32),
            pltpu.SMEM((64,), jnp.int32),
            pltpu.VMEM((4096,), jnp.float32),
            pltpu.VMEM((CHUNK,), jnp.int32),
            pltpu.VMEM((CHUNK,), jnp.float32),
            pltpu.VMEM_SHARED((BSZ,), jnp.float32),
            pltpu.SemaphoreType.DMA,
        ],
    )(bidx, bval, hist)

    return out1.reshape(B_, H_, W_, C_)
